# baseline (speedup 1.0000x reference)
import functools

import jax
import jax.numpy as jnp
from jax import lax
from jax.experimental import pallas as pl
from jax.experimental.pallas import tpu as pltpu
from jax.experimental.pallas import tpu_sc as plsc

EPS = 0.1
WIN = 32
NW = 32
BAND = WIN + 128


def _sc_weights(labels_pad_flat, B, S):
    R = B * S
    RPW = R // NW
    LABW = WIN + RPW + WIN
    SP = S + WIN
    BIG = jnp.int32(1 << 20)

    mesh = plsc.VectorSubcoreMesh(core_axis_name="c", subcore_axis_name="s")

    @functools.partial(
        pl.kernel,
        mesh=mesh,
        out_type=jax.ShapeDtypeStruct((NW, RPW, BAND), jnp.float32),
        scratch_types=[
            pltpu.VMEM((LABW,), jnp.int32),
            pltpu.VMEM((WIN + RPW,), jnp.int32),
            pltpu.VMEM((RPW, BAND), jnp.float32),
        ],
    )
    def sc_kernel(labpad_hbm, wout_hbm, lab_v, nxt_v, w_v):
        wid = lax.axis_index("s") * 2 + lax.axis_index("c")
        r0 = wid * RPW
        b = r0 // S
        i0 = r0 - b * S
        win_start = b * SP + i0
        pltpu.sync_copy(labpad_hbm.at[pl.ds(win_start, LABW)], lab_v)

        lane = lax.iota(jnp.int32, 16)

        for c in range((WIN + RPW) // 16):
            zc = lab_v[pl.ds(c * 16, 16)]
            nxt = jnp.full((16,), BIG, jnp.int32)
            for d in range(WIN - 1, 0, -1):
                zd = lab_v[pl.ds(c * 16 + d, 16)]
                nxt = jnp.where(zc == zd, jnp.int32(d), nxt)
            nxt_v[pl.ds(c * 16, 16)] = nxt

        one16 = jnp.ones((16,), jnp.float32)
        zero16 = jnp.zeros((16,), jnp.float32)

        for row in range(RPW):
            for h in range(2):
                start = row + 1 + h * 16
                z = lab_v[pl.ds(start, 16)]
                nx = nxt_v[pl.ds(start, 16)]
                avec = lane + start
                keep = (z != 0) & (nx + avec > row + WIN)
                w_v[row, pl.ds(start, 16)] = jnp.where(keep, one16, zero16)

        pltpu.sync_copy(w_v, wout_hbm.at[wid])

    return sc_kernel(labels_pad_flat)


def _tc_main(logits, labels_pad_col, wband, B, S, V):
    R = B * S
    RPW = R // NW
    RPT = 2 * RPW
    TS = S // RPT
    SP = S + WIN

    def body(xref_a, xref_b, labref, wref, oref, acc):
        bi = pl.program_id(0)
        ti = pl.program_id(1)

        @pl.when((bi == 0) & (ti == 0))
        def _init():
            acc[0] = 0.0
            acc[1] = 0.0

        tcol = lax.broadcasted_iota(jnp.int32, (RPW, 1), 0)
        jlane = lax.broadcasted_iota(jnp.int32, (1, BAND), 1)
        km = tcol + WIN - jlane
        band = (km >= 1) & (km <= WIN - 1)
        vio = lax.broadcasted_iota(jnp.int32, (1, V), 1)

        ce_part = jnp.float32(0.0)
        u_part = jnp.float32(0.0)
        for op in range(2):
            x = (xref_a if op == 0 else xref_b)[0]
            m1 = jnp.max(x, axis=1, keepdims=True)
            s1 = jnp.sum(jnp.exp(x - m1), axis=1, keepdims=True)
            lse1 = m1 + jnp.log(s1)
            mean1 = jnp.sum(x, axis=1, keepdims=True) * (1.0 / V)
            xb = x.astype(jnp.bfloat16)

            for h in range(2):
                seg = op * 2 + h
                woff = bi * SP + ti * (4 * RPW) + seg * RPW
                labwin = labref[pl.ds(woff, BAND), :]
                hot = (labwin == vio).astype(jnp.bfloat16)
                G = lax.dot_general(
                    xb[h * RPW:(h + 1) * RPW], hot,
                    (((1,), (1,)), ((), ())),
                    preferred_element_type=jnp.float32)

                lse_h = lse1[h * RPW:(h + 1) * RPW]
                mean_h = mean1[h * RPW:(h + 1) * RPW]

                p = jnp.exp(G - lse_h)
                ue = -jnp.log(jnp.maximum(1.0 - p, 1e-5))
                wb = wref[seg]
                u_part += jnp.sum(jnp.where(band, wb * ue, 0.0))

                g0 = jnp.sum(jnp.where(km == 0, G, 0.0),
                             axis=1, keepdims=True)
                ce_part += jnp.sum(lse_h - (1.0 - EPS) * g0 - EPS * mean_h)

        acc[0] += ce_part
        acc[1] += u_part

        @pl.when((bi == B - 1) & (ti == TS - 1))
        def _fin():
            oref[0, 0] = acc[0] / R + jnp.log(1.0 + acc[1] / B)

    out = pl.pallas_call(
        body,
        grid=(B, TS // 2),
        in_specs=[
            pl.BlockSpec((1, RPT, V), lambda b, t: (b, 2 * t, 0)),
            pl.BlockSpec((1, RPT, V), lambda b, t: (b, 2 * t + 1, 0)),
            pl.BlockSpec((B * SP, 1), lambda b, t: (0, 0)),
            pl.BlockSpec((4, RPW, BAND), lambda b, t: (b * TS // 2 + t, 0, 0)),
        ],
        out_specs=pl.BlockSpec(memory_space=pltpu.SMEM),
        out_shape=jax.ShapeDtypeStruct((1, 1), jnp.float32),
        scratch_shapes=[pltpu.SMEM((2,), jnp.float32)],
    )(logits, logits, labels_pad_col, wband)
    return out[0, 0]


def kernel(logits, labels):
    B, S, V = logits.shape
    labels_pad = jnp.pad(labels, ((0, 0), (WIN, 0)))
    flat_sc = jnp.pad(labels_pad.reshape(-1), (0, WIN))
    wband = _sc_weights(flat_sc, B, S)
    return _tc_main(logits, labels_pad.reshape(-1, 1), wband, B, S, V)
